# baseline (speedup 1.0000x reference)
import functools
import jax
import jax.numpy as jnp
from jax import lax
from jax.experimental import pallas as pl
from jax.experimental.pallas import tpu as pltpu
from jax.experimental.pallas import tpu_sc as plsc

L = 16
IT = 16
JV = 4
JT = JV * L
_INF = float("inf")


def _make_sc_kernel(B, N, M, NQ):
    NS = 8 * NQ
    NIT = NQ // IT
    NJT = M // JT
    mesh = plsc.VectorSubcoreMesh(core_axis_name="c", subcore_axis_name="s")

    @functools.partial(
        pl.kernel,
        mesh=mesh,
        out_type=[
            jax.ShapeDtypeStruct((B * NS,), jnp.float32),
            jax.ShapeDtypeStruct((B * M,), jnp.float32),
        ],
        scratch_types=[
            pltpu.VMEM((M,), jnp.float32),
            pltpu.VMEM((M,), jnp.float32),
            pltpu.VMEM((M,), jnp.float32),
            pltpu.VMEM((M,), jnp.float32),
            pltpu.VMEM((NQ,), jnp.float32),
            pltpu.VMEM((NQ,), jnp.float32),
            pltpu.VMEM((NQ,), jnp.float32),
            pltpu.VMEM((NQ,), jnp.float32),
            pltpu.VMEM((512,), jnp.float32),
            pltpu.VMEM((M,), jnp.float32),
            pltpu.VMEM((8 * 512,), jnp.float32),
            pltpu.VMEM((NQ * 16,), jnp.float32),
            pltpu.VMEM((NQ * 16,), jnp.float32),
            pltpu.VMEM((NQ * 16,), jnp.float32),
            pltpu.VMEM((NQ * 16,), jnp.float32),
            pltpu.VMEM_SHARED((16 * M,), jnp.float32),
        ],
    )
    def sc_kernel(x1t, x2t, out1, out2,
                  rx_ref, ry_ref, rz_ref, rn_ref,
                  qsx_ref, qsy_ref, qsz_ref, qn_ref, d1_ref, d2_ref,
                  mrg_ref, px_ref, py_ref, pz_ref, pn_ref, shared):
        c = lax.axis_index("c")
        s = lax.axis_index("s")
        b = c * 2 + (s >> 3)
        q0 = (s & 7) * NQ

        r_refs = (rx_ref, ry_ref, rz_ref)
        q_refs = (qsx_ref, qsy_ref, qsz_ref)
        for row in range(3):
            pltpu.sync_copy(x2t.at[pl.ds((b * 3 + row) * M, M)], r_refs[row])
            pltpu.sync_copy(x1t.at[pl.ds((b * 3 + row) * N + q0, NQ)],
                            q_refs[row])

        def rn_body(v, carry):
            sl = pl.ds(v * L, L)
            x = rx_ref[sl]
            y = ry_ref[sl]
            z = rz_ref[sl]
            rn_ref[sl] = x * x + y * y + z * z
            return carry

        lax.fori_loop(0, M // L, rn_body, 0)

        inf_vec = jnp.full((L,), _INF, jnp.float32)

        def qn_body(v, carry):
            sl = pl.ds(v * L, L)
            x = qsx_ref[sl]
            y = qsy_ref[sl]
            z = qsz_ref[sl]
            qn_ref[sl] = x * x + y * y + z * z
            qsx_ref[sl] = x * -2.0
            qsy_ref[sl] = y * -2.0
            qsz_ref[sl] = z * -2.0
            return carry

        lax.fori_loop(0, NQ // L, qn_body, 0)

        def d2init_body(v, carry):
            d2_ref[pl.ds(v * L, L)] = inf_vec
            return carry

        lax.fori_loop(0, M // L, d2init_body, 0)

        lane = lax.iota(jnp.int32, L)
        _dn = lax.GatherDimensionNumbers(offset_dims=(),
                                         collapsed_slice_dims=(0,),
                                         start_index_map=(0,))

        def _shuf(v, idx):
            return lax.gather(v, idx[:, None], _dn, (1,),
                              mode=lax.GatherScatterMode.PROMISE_IN_BOUNDS)

        idxs = {k: lane ^ k for k in (8, 4, 2, 1)}
        sels = {k: (lane & k) == 0 for k in (8, 4, 2, 1)}
        bitrev = (((lane & 1) << 3) | ((lane & 2) << 1)
                  | ((lane & 4) >> 1) | ((lane & 8) >> 3))

        def _hmin16(vecs):
            for k in (8, 4, 2, 1):
                vecs = [jnp.minimum(v, _shuf(v, idxs[k])) for v in vecs]
                vecs = [jnp.where(sels[k], vecs[2 * i], vecs[2 * i + 1])
                        for i in range(len(vecs) // 2)]
            return _shuf(vecs[0], bitrev)

        lane_consts = [lane * 0 + ii for ii in range(L)]

        def splat_body(iv, carry):
            base = iv * L
            xs = qsx_ref[pl.ds(base, L)]
            ys = qsy_ref[pl.ds(base, L)]
            zs = qsz_ref[pl.ds(base, L)]
            ns = qn_ref[pl.ds(base, L)]
            for ii in range(L):
                o = pl.ds((base + ii) * L, L)
                px_ref[o] = _shuf(xs, lane_consts[ii])
                py_ref[o] = _shuf(ys, lane_consts[ii])
                pz_ref[o] = _shuf(zs, lane_consts[ii])
                pn_ref[o] = _shuf(ns, lane_consts[ii])
            return carry

        lax.fori_loop(0, NQ // L, splat_body, 0)

        def it_body(it, carry):
            i0 = it * IT
            qnv = qn_ref[pl.ds(i0, IT)]

            def jt_body(jt, smin):
                j0 = jt * JT
                rx = [rx_ref[pl.ds(j0 + v * L, L)] for v in range(JV)]
                ry = [ry_ref[pl.ds(j0 + v * L, L)] for v in range(JV)]
                rz = [rz_ref[pl.ds(j0 + v * L, L)] for v in range(JV)]
                rn = [rn_ref[pl.ds(j0 + v * L, L)] for v in range(JV)]
                d2a = [d2_ref[pl.ds(j0 + v * L, L)] for v in range(JV)]
                smin = list(smin)
                for ii in range(IT):
                    o = pl.ds((i0 + ii) * L, L)
                    qx = px_ref[o]
                    qy = py_ref[o]
                    qz = pz_ref[o]
                    qn = pn_ref[o]
                    for v in range(JV):
                        t = rx[v] * qx + ry[v] * qy + rz[v] * qz
                        smin[ii] = jnp.minimum(smin[ii], t + rn[v])
                        d2a[v] = jnp.minimum(d2a[v], t + qn)
                for v in range(JV):
                    d2_ref[pl.ds(j0 + v * L, L)] = d2a[v]
                return tuple(smin)

            smin = lax.fori_loop(0, NJT, jt_body, (inf_vec,) * IT)
            d1_ref[pl.ds(i0, IT)] = _hmin16(list(smin)) + qnv
            return carry

        lax.fori_loop(0, NIT, it_body, 0)

        pltpu.sync_copy(d1_ref.at[pl.ds(0, NQ)], out1.at[pl.ds(b * NS + q0, NQ)])
        pltpu.sync_copy(d2_ref, shared.at[pl.ds(s * M, M)])
        plsc.subcore_barrier()

        MW = M // 8
        base_row = (s >> 3) << 3
        jL = (s & 7) * MW
        for r in range(8):
            pltpu.sync_copy(shared.at[pl.ds((base_row + r) * M + jL, MW)],
                            mrg_ref.at[pl.ds(r * MW, MW)])

        def mrg_body(v, carry):
            sl = pl.ds(v * L, L)
            acc = mrg_ref[pl.ds(0 * MW + v * L, L)]
            for r in range(1, 8):
                acc = jnp.minimum(acc, mrg_ref[pl.ds(r * MW + v * L, L)])
            d1_ref[sl] = acc + rn_ref[pl.ds(jL + v * L, L)]
            return carry

        lax.fori_loop(0, MW // L, mrg_body, 0)
        pltpu.sync_copy(d1_ref.at[pl.ds(0, MW)], out2.at[pl.ds(b * M + jL, MW)])

    return sc_kernel


NSC = 1024
BN = 512


def _tc_body(x_ref, yt_ref, out1_ref, out2_ref):
    ib = pl.program_id(1)
    x = x_ref[0, :, :]
    yt = yt_ref[0, :, :]
    acc = None
    for cc in range(3):
        d = x[:, cc:cc + 1] - yt[cc:cc + 1, :]
        acc = d * d if acc is None else acc + d * d
    out1_ref[0, 0, pl.ds(ib * BN, BN)] = jnp.min(acc, axis=1)
    part2 = jnp.min(acc, axis=0)

    @pl.when(ib == 0)
    def _():
        out2_ref[0, 0, :] = part2

    @pl.when(ib != 0)
    def _():
        out2_ref[0, 0, :] = jnp.minimum(out2_ref[0, 0, :], part2)


def _merge_body(a_ref, b_ref, o_ref):
    o_ref[...] = jnp.minimum(a_ref[...], b_ref[...])


def kernel(xyz1, xyz2):
    B, N, _ = xyz1.shape
    M = xyz2.shape[1]
    x1t = jnp.transpose(xyz1, (0, 2, 1)).reshape(B * 3 * N)
    x2t = jnp.transpose(xyz2, (0, 2, 1)).reshape(B * 3 * M)
    sc_d1, sc_d2 = _make_sc_kernel(B, N, M, NSC // 8)(x1t, x2t)

    yt = jnp.transpose(xyz2, (0, 2, 1))
    NT = N - NSC
    tc_d1, tc_d2 = pl.pallas_call(
        _tc_body,
        grid=(B, NT // BN),
        in_specs=[
            pl.BlockSpec((1, BN, 3), lambda b, i: (b, i, 0)),
            pl.BlockSpec((1, 3, M), lambda b, i: (b, 0, 0)),
        ],
        out_specs=[
            pl.BlockSpec((1, 1, NT), lambda b, i: (b, 0, 0)),
            pl.BlockSpec((1, 1, M), lambda b, i: (b, 0, 0)),
        ],
        out_shape=[
            jax.ShapeDtypeStruct((B, 1, NT), jnp.float32),
            jax.ShapeDtypeStruct((B, 1, M), jnp.float32),
        ],
        compiler_params=pltpu.CompilerParams(
            dimension_semantics=("parallel", "arbitrary"),
        ),
    )(xyz1[:, NSC:], yt)

    dist2 = pl.pallas_call(
        _merge_body,
        out_shape=jax.ShapeDtypeStruct((B, M), jnp.float32),
    )(tc_d2.reshape(B, M), sc_d2.reshape(B, M))
    dist1 = jnp.concatenate(
        [sc_d1.reshape(B, NSC), tc_d1.reshape(B, NT)], axis=1)
    return (dist1, dist2)


# speedup vs baseline: 1.0844x; 1.0844x over previous
import functools
import jax
import jax.numpy as jnp
from jax import lax
from jax.experimental import pallas as pl
from jax.experimental.pallas import tpu as pltpu
from jax.experimental.pallas import tpu_sc as plsc

L = 16
IT = 16
JV = 4
JT = JV * L
_INF = float("inf")


def _make_sc_kernel(B, N, M, NQ):
    NS = 8 * NQ
    NIT = NQ // IT
    NJT = M // JT
    mesh = plsc.VectorSubcoreMesh(core_axis_name="c", subcore_axis_name="s")

    @functools.partial(
        pl.kernel,
        mesh=mesh,
        out_type=[
            jax.ShapeDtypeStruct((B * NS,), jnp.float32),
            jax.ShapeDtypeStruct((B * M,), jnp.float32),
        ],
        scratch_types=[
            pltpu.VMEM((M,), jnp.float32),
            pltpu.VMEM((M,), jnp.float32),
            pltpu.VMEM((M,), jnp.float32),
            pltpu.VMEM((M,), jnp.float32),
            pltpu.VMEM((NQ,), jnp.float32),
            pltpu.VMEM((NQ,), jnp.float32),
            pltpu.VMEM((NQ,), jnp.float32),
            pltpu.VMEM((NQ,), jnp.float32),
            pltpu.VMEM((512,), jnp.float32),
            pltpu.VMEM((M,), jnp.float32),
            pltpu.VMEM((8 * 512,), jnp.float32),
            pltpu.VMEM((NQ * 16,), jnp.float32),
            pltpu.VMEM((NQ * 16,), jnp.float32),
            pltpu.VMEM((NQ * 16,), jnp.float32),
            pltpu.VMEM((NQ * 16,), jnp.float32),
            pltpu.VMEM_SHARED((16 * M,), jnp.float32),
        ],
    )
    def sc_kernel(x1t, x2t, out1, out2,
                  rx_ref, ry_ref, rz_ref, rn_ref,
                  qsx_ref, qsy_ref, qsz_ref, qn_ref, d1_ref, d2_ref,
                  mrg_ref, px_ref, py_ref, pz_ref, pn_ref, shared):
        c = lax.axis_index("c")
        s = lax.axis_index("s")
        b = c * 2 + (s >> 3)
        q0 = (s & 7) * NQ

        r_refs = (rx_ref, ry_ref, rz_ref)
        q_refs = (qsx_ref, qsy_ref, qsz_ref)
        for row in range(3):
            pltpu.sync_copy(x2t.at[pl.ds((b * 3 + row) * M, M)], r_refs[row])
            pltpu.sync_copy(x1t.at[pl.ds((b * 3 + row) * N + q0, NQ)],
                            q_refs[row])

        def rn_body(v, carry):
            sl = pl.ds(v * L, L)
            x = rx_ref[sl]
            y = ry_ref[sl]
            z = rz_ref[sl]
            rn_ref[sl] = x * x + y * y + z * z
            return carry

        lax.fori_loop(0, M // L, rn_body, 0)

        inf_vec = jnp.full((L,), _INF, jnp.float32)

        def qn_body(v, carry):
            sl = pl.ds(v * L, L)
            x = qsx_ref[sl]
            y = qsy_ref[sl]
            z = qsz_ref[sl]
            qn_ref[sl] = x * x + y * y + z * z
            qsx_ref[sl] = x * -2.0
            qsy_ref[sl] = y * -2.0
            qsz_ref[sl] = z * -2.0
            return carry

        lax.fori_loop(0, NQ // L, qn_body, 0)

        def d2init_body(v, carry):
            d2_ref[pl.ds(v * L, L)] = inf_vec
            return carry

        lax.fori_loop(0, M // L, d2init_body, 0)

        lane = lax.iota(jnp.int32, L)
        _dn = lax.GatherDimensionNumbers(offset_dims=(),
                                         collapsed_slice_dims=(0,),
                                         start_index_map=(0,))

        def _shuf(v, idx):
            return lax.gather(v, idx[:, None], _dn, (1,),
                              mode=lax.GatherScatterMode.PROMISE_IN_BOUNDS)

        idxs = {k: lane ^ k for k in (8, 4, 2, 1)}
        sels = {k: (lane & k) == 0 for k in (8, 4, 2, 1)}
        bitrev = (((lane & 1) << 3) | ((lane & 2) << 1)
                  | ((lane & 4) >> 1) | ((lane & 8) >> 3))

        def _hmin16(vecs):
            for k in (8, 4, 2, 1):
                vecs = [jnp.minimum(v, _shuf(v, idxs[k])) for v in vecs]
                vecs = [jnp.where(sels[k], vecs[2 * i], vecs[2 * i + 1])
                        for i in range(len(vecs) // 2)]
            return _shuf(vecs[0], bitrev)

        lane_consts = [lane * 0 + ii for ii in range(L)]

        def splat_body(iv, carry):
            base = iv * L
            xs = qsx_ref[pl.ds(base, L)]
            ys = qsy_ref[pl.ds(base, L)]
            zs = qsz_ref[pl.ds(base, L)]
            ns = qn_ref[pl.ds(base, L)]
            for ii in range(L):
                o = pl.ds((base + ii) * L, L)
                px_ref[o] = _shuf(xs, lane_consts[ii])
                py_ref[o] = _shuf(ys, lane_consts[ii])
                pz_ref[o] = _shuf(zs, lane_consts[ii])
                pn_ref[o] = _shuf(ns, lane_consts[ii])
            return carry

        lax.fori_loop(0, NQ // L, splat_body, 0)

        def it_body(it, carry):
            i0 = it * IT
            qnv = qn_ref[pl.ds(i0, IT)]

            def jt_body(jt, smin):
                j0 = jt * JT
                rx = [rx_ref[pl.ds(j0 + v * L, L)] for v in range(JV)]
                ry = [ry_ref[pl.ds(j0 + v * L, L)] for v in range(JV)]
                rz = [rz_ref[pl.ds(j0 + v * L, L)] for v in range(JV)]
                rn = [rn_ref[pl.ds(j0 + v * L, L)] for v in range(JV)]
                d2a = [d2_ref[pl.ds(j0 + v * L, L)] for v in range(JV)]
                smin = list(smin)
                for ii in range(IT):
                    o = pl.ds((i0 + ii) * L, L)
                    qx = px_ref[o]
                    qy = py_ref[o]
                    qz = pz_ref[o]
                    qn = pn_ref[o]
                    for v in range(JV):
                        t = rx[v] * qx + ry[v] * qy + rz[v] * qz
                        smin[ii] = jnp.minimum(smin[ii], t + rn[v])
                        d2a[v] = jnp.minimum(d2a[v], t + qn)
                for v in range(JV):
                    d2_ref[pl.ds(j0 + v * L, L)] = d2a[v]
                return tuple(smin)

            smin = lax.fori_loop(0, NJT, jt_body, (inf_vec,) * IT)
            d1_ref[pl.ds(i0, IT)] = _hmin16(list(smin)) + qnv
            return carry

        lax.fori_loop(0, NIT, it_body, 0)

        pltpu.sync_copy(d1_ref.at[pl.ds(0, NQ)], out1.at[pl.ds(b * NS + q0, NQ)])
        pltpu.sync_copy(d2_ref, shared.at[pl.ds(s * M, M)])
        plsc.subcore_barrier()

        MW = M // 8
        base_row = (s >> 3) << 3
        jL = (s & 7) * MW
        for r in range(8):
            pltpu.sync_copy(shared.at[pl.ds((base_row + r) * M + jL, MW)],
                            mrg_ref.at[pl.ds(r * MW, MW)])

        def mrg_body(v, carry):
            sl = pl.ds(v * L, L)
            acc = mrg_ref[pl.ds(0 * MW + v * L, L)]
            for r in range(1, 8):
                acc = jnp.minimum(acc, mrg_ref[pl.ds(r * MW + v * L, L)])
            d1_ref[sl] = acc + rn_ref[pl.ds(jL + v * L, L)]
            return carry

        lax.fori_loop(0, MW // L, mrg_body, 0)
        pltpu.sync_copy(d1_ref.at[pl.ds(0, MW)], out2.at[pl.ds(b * M + jL, MW)])

    return sc_kernel


NSC = 1024
BN = 1024


def _tc_body(x_ref, yt_ref, out1_ref, out2_ref):
    ib = pl.program_id(1)
    x = x_ref[0, :, :]
    yt = yt_ref[0, :, :]
    qn = jnp.sum(x * x, axis=1, keepdims=True)
    rn = jnp.sum(yt * yt, axis=0, keepdims=True)
    acc = rn
    for cc in range(3):
        xc = x[:, cc:cc + 1] * -2.0
        acc = acc + xc * yt[cc:cc + 1, :]
    out1_ref[0, 0, pl.ds(ib * BN, BN)] = jnp.min(acc, axis=1) + qn[:, 0]
    part2 = jnp.min(acc + qn, axis=0)

    @pl.when(ib == 0)
    def _():
        out2_ref[0, 0, :] = part2

    @pl.when(ib != 0)
    def _():
        out2_ref[0, 0, :] = jnp.minimum(out2_ref[0, 0, :], part2)


def _merge_body(a_ref, b_ref, o_ref):
    o_ref[...] = jnp.minimum(a_ref[...], b_ref[...])


def kernel(xyz1, xyz2):
    B, N, _ = xyz1.shape
    M = xyz2.shape[1]
    x1t = jnp.transpose(xyz1, (0, 2, 1)).reshape(B * 3 * N)
    x2t = jnp.transpose(xyz2, (0, 2, 1)).reshape(B * 3 * M)
    sc_d1, sc_d2 = _make_sc_kernel(B, N, M, NSC // 8)(x1t, x2t)

    yt = jnp.transpose(xyz2, (0, 2, 1))
    NT = N - NSC
    tc_d1, tc_d2 = pl.pallas_call(
        _tc_body,
        grid=(B, NT // BN),
        in_specs=[
            pl.BlockSpec((1, BN, 3), lambda b, i: (b, i, 0)),
            pl.BlockSpec((1, 3, M), lambda b, i: (b, 0, 0)),
        ],
        out_specs=[
            pl.BlockSpec((1, 1, NT), lambda b, i: (b, 0, 0)),
            pl.BlockSpec((1, 1, M), lambda b, i: (b, 0, 0)),
        ],
        out_shape=[
            jax.ShapeDtypeStruct((B, 1, NT), jnp.float32),
            jax.ShapeDtypeStruct((B, 1, M), jnp.float32),
        ],
        compiler_params=pltpu.CompilerParams(
            dimension_semantics=("parallel", "arbitrary"),
        ),
    )(xyz1[:, NSC:], yt)

    dist2 = pl.pallas_call(
        _merge_body,
        out_shape=jax.ShapeDtypeStruct((B, M), jnp.float32),
    )(tc_d2.reshape(B, M), sc_d2.reshape(B, M))
    dist1 = jnp.concatenate(
        [sc_d1.reshape(B, NSC), tc_d1.reshape(B, NT)], axis=1)
    return (dist1, dist2)


# speedup vs baseline: 1.0883x; 1.0036x over previous
import functools
import jax
import jax.numpy as jnp
from jax import lax
from jax.experimental import pallas as pl
from jax.experimental.pallas import tpu as pltpu
from jax.experimental.pallas import tpu_sc as plsc

L = 16
IT = 16
JV = 4
JT = JV * L
_INF = float("inf")


def _make_sc_kernel(B, N, M, NQ):
    NS = 8 * NQ
    NIT = NQ // IT
    NJT = M // JT
    mesh = plsc.VectorSubcoreMesh(core_axis_name="c", subcore_axis_name="s")

    @functools.partial(
        pl.kernel,
        mesh=mesh,
        out_type=[
            jax.ShapeDtypeStruct((B * NS,), jnp.float32),
            jax.ShapeDtypeStruct((B * M,), jnp.float32),
        ],
        scratch_types=[
            pltpu.VMEM((M,), jnp.float32),
            pltpu.VMEM((M,), jnp.float32),
            pltpu.VMEM((M,), jnp.float32),
            pltpu.VMEM((M,), jnp.float32),
            pltpu.VMEM((NQ,), jnp.float32),
            pltpu.VMEM((NQ,), jnp.float32),
            pltpu.VMEM((NQ,), jnp.float32),
            pltpu.VMEM((NQ,), jnp.float32),
            pltpu.VMEM((512,), jnp.float32),
            pltpu.VMEM((M,), jnp.float32),
            pltpu.VMEM((8 * 512,), jnp.float32),
            pltpu.VMEM((NQ * 16,), jnp.float32),
            pltpu.VMEM((NQ * 16,), jnp.float32),
            pltpu.VMEM((NQ * 16,), jnp.float32),
            pltpu.VMEM((NQ * 16,), jnp.float32),
            pltpu.VMEM_SHARED((16 * M,), jnp.float32),
        ],
    )
    def sc_kernel(x1t, x2t, out1, out2,
                  rx_ref, ry_ref, rz_ref, rn_ref,
                  qsx_ref, qsy_ref, qsz_ref, qn_ref, d1_ref, d2_ref,
                  mrg_ref, px_ref, py_ref, pz_ref, pn_ref, shared):
        c = lax.axis_index("c")
        s = lax.axis_index("s")
        b = c * 2 + (s >> 3)
        q0 = (s & 7) * NQ

        r_refs = (rx_ref, ry_ref, rz_ref)
        q_refs = (qsx_ref, qsy_ref, qsz_ref)
        for row in range(3):
            pltpu.sync_copy(x2t.at[pl.ds((b * 3 + row) * M, M)], r_refs[row])
            pltpu.sync_copy(x1t.at[pl.ds((b * 3 + row) * N + q0, NQ)],
                            q_refs[row])

        def rn_body(v, carry):
            sl = pl.ds(v * L, L)
            x = rx_ref[sl]
            y = ry_ref[sl]
            z = rz_ref[sl]
            rn_ref[sl] = x * x + y * y + z * z
            return carry

        lax.fori_loop(0, M // L, rn_body, 0)

        inf_vec = jnp.full((L,), _INF, jnp.float32)

        def qn_body(v, carry):
            sl = pl.ds(v * L, L)
            x = qsx_ref[sl]
            y = qsy_ref[sl]
            z = qsz_ref[sl]
            qn_ref[sl] = x * x + y * y + z * z
            qsx_ref[sl] = x * -2.0
            qsy_ref[sl] = y * -2.0
            qsz_ref[sl] = z * -2.0
            return carry

        lax.fori_loop(0, NQ // L, qn_body, 0)

        def d2init_body(v, carry):
            d2_ref[pl.ds(v * L, L)] = inf_vec
            return carry

        lax.fori_loop(0, M // L, d2init_body, 0)

        lane = lax.iota(jnp.int32, L)
        _dn = lax.GatherDimensionNumbers(offset_dims=(),
                                         collapsed_slice_dims=(0,),
                                         start_index_map=(0,))

        def _shuf(v, idx):
            return lax.gather(v, idx[:, None], _dn, (1,),
                              mode=lax.GatherScatterMode.PROMISE_IN_BOUNDS)

        idxs = {k: lane ^ k for k in (8, 4, 2, 1)}
        sels = {k: (lane & k) == 0 for k in (8, 4, 2, 1)}
        bitrev = (((lane & 1) << 3) | ((lane & 2) << 1)
                  | ((lane & 4) >> 1) | ((lane & 8) >> 3))

        def _hmin16(vecs):
            for k in (8, 4, 2, 1):
                vecs = [jnp.minimum(v, _shuf(v, idxs[k])) for v in vecs]
                vecs = [jnp.where(sels[k], vecs[2 * i], vecs[2 * i + 1])
                        for i in range(len(vecs) // 2)]
            return _shuf(vecs[0], bitrev)

        lane_consts = [lane * 0 + ii for ii in range(L)]

        def splat_body(iv, carry):
            base = iv * L
            xs = qsx_ref[pl.ds(base, L)]
            ys = qsy_ref[pl.ds(base, L)]
            zs = qsz_ref[pl.ds(base, L)]
            ns = qn_ref[pl.ds(base, L)]
            for ii in range(L):
                o = pl.ds((base + ii) * L, L)
                px_ref[o] = _shuf(xs, lane_consts[ii])
                py_ref[o] = _shuf(ys, lane_consts[ii])
                pz_ref[o] = _shuf(zs, lane_consts[ii])
                pn_ref[o] = _shuf(ns, lane_consts[ii])
            return carry

        lax.fori_loop(0, NQ // L, splat_body, 0)

        def it_body(it, carry):
            i0 = it * IT
            qnv = qn_ref[pl.ds(i0, IT)]

            def jt_body(jt, smin):
                j0 = jt * JT
                rx = [rx_ref[pl.ds(j0 + v * L, L)] for v in range(JV)]
                ry = [ry_ref[pl.ds(j0 + v * L, L)] for v in range(JV)]
                rz = [rz_ref[pl.ds(j0 + v * L, L)] for v in range(JV)]
                rn = [rn_ref[pl.ds(j0 + v * L, L)] for v in range(JV)]
                d2a = [d2_ref[pl.ds(j0 + v * L, L)] for v in range(JV)]
                smin = list(smin)
                for ii in range(IT):
                    o = pl.ds((i0 + ii) * L, L)
                    qx = px_ref[o]
                    qy = py_ref[o]
                    qz = pz_ref[o]
                    qn = pn_ref[o]
                    for v in range(JV):
                        t = rx[v] * qx + ry[v] * qy + rz[v] * qz
                        smin[ii] = jnp.minimum(smin[ii], t + rn[v])
                        d2a[v] = jnp.minimum(d2a[v], t + qn)
                for v in range(JV):
                    d2_ref[pl.ds(j0 + v * L, L)] = d2a[v]
                return tuple(smin)

            smin = lax.fori_loop(0, NJT, jt_body, (inf_vec,) * IT)
            d1_ref[pl.ds(i0, IT)] = _hmin16(list(smin)) + qnv
            return carry

        lax.fori_loop(0, NIT, it_body, 0)

        pltpu.sync_copy(d1_ref.at[pl.ds(0, NQ)], out1.at[pl.ds(b * NS + q0, NQ)])
        pltpu.sync_copy(d2_ref, shared.at[pl.ds(s * M, M)])
        plsc.subcore_barrier()

        MW = M // 8
        base_row = (s >> 3) << 3
        jL = (s & 7) * MW
        for r in range(8):
            pltpu.sync_copy(shared.at[pl.ds((base_row + r) * M + jL, MW)],
                            mrg_ref.at[pl.ds(r * MW, MW)])

        def mrg_body(v, carry):
            sl = pl.ds(v * L, L)
            acc = mrg_ref[pl.ds(0 * MW + v * L, L)]
            for r in range(1, 8):
                acc = jnp.minimum(acc, mrg_ref[pl.ds(r * MW + v * L, L)])
            d1_ref[sl] = acc + rn_ref[pl.ds(jL + v * L, L)]
            return carry

        lax.fori_loop(0, MW // L, mrg_body, 0)
        pltpu.sync_copy(d1_ref.at[pl.ds(0, MW)], out2.at[pl.ds(b * M + jL, MW)])

    return sc_kernel


NSC = 1024
BN = 1536


def _tc_body(x_ref, yt_ref, out1_ref, out2_ref):
    ib = pl.program_id(1)
    x = x_ref[0, :, :]
    yt = yt_ref[0, :, :]
    qn = jnp.sum(x * x, axis=1, keepdims=True)
    rn = jnp.sum(yt * yt, axis=0, keepdims=True)
    acc = rn
    for cc in range(3):
        xc = x[:, cc:cc + 1] * -2.0
        acc = acc + xc * yt[cc:cc + 1, :]
    out1_ref[0, 0, pl.ds(ib * BN, BN)] = jnp.min(acc, axis=1) + qn[:, 0]
    part2 = jnp.min(acc + qn, axis=0)

    @pl.when(ib == 0)
    def _():
        out2_ref[0, 0, :] = part2

    @pl.when(ib != 0)
    def _():
        out2_ref[0, 0, :] = jnp.minimum(out2_ref[0, 0, :], part2)


def _merge_body(a_ref, b_ref, o_ref):
    o_ref[...] = jnp.minimum(a_ref[...], b_ref[...])


def kernel(xyz1, xyz2):
    B, N, _ = xyz1.shape
    M = xyz2.shape[1]
    x1t = jnp.transpose(xyz1, (0, 2, 1)).reshape(B * 3 * N)
    x2t = jnp.transpose(xyz2, (0, 2, 1)).reshape(B * 3 * M)
    sc_d1, sc_d2 = _make_sc_kernel(B, N, M, NSC // 8)(x1t, x2t)

    yt = jnp.transpose(xyz2, (0, 2, 1))
    NT = N - NSC
    tc_d1, tc_d2 = pl.pallas_call(
        _tc_body,
        grid=(B, NT // BN),
        in_specs=[
            pl.BlockSpec((1, BN, 3), lambda b, i: (b, i, 0)),
            pl.BlockSpec((1, 3, M), lambda b, i: (b, 0, 0)),
        ],
        out_specs=[
            pl.BlockSpec((1, 1, NT), lambda b, i: (b, 0, 0)),
            pl.BlockSpec((1, 1, M), lambda b, i: (b, 0, 0)),
        ],
        out_shape=[
            jax.ShapeDtypeStruct((B, 1, NT), jnp.float32),
            jax.ShapeDtypeStruct((B, 1, M), jnp.float32),
        ],
        compiler_params=pltpu.CompilerParams(
            dimension_semantics=("parallel", "arbitrary"),
        ),
    )(xyz1[:, NSC:], yt)

    dist2 = pl.pallas_call(
        _merge_body,
        out_shape=jax.ShapeDtypeStruct((B, M), jnp.float32),
    )(tc_d2.reshape(B, M), sc_d2.reshape(B, M))
    dist1 = jnp.concatenate(
        [sc_d1.reshape(B, NSC), tc_d1.reshape(B, NT)], axis=1)
    return (dist1, dist2)
